# baseline (speedup 1.0000x reference)
import jax
import jax.numpy as jnp
from jax import lax
from jax.experimental import pallas as pl
from jax.experimental.pallas import tpu as pltpu

B, S, D, R, N_INPUT, N_PROCESS, K = 4, 2048, 2048, 256, 8, 32, 8
T_BLK = 512


def _fused_kernel(x_ref, w_ref, pidx_ref, wn_ref, p_ref, out_ref, wscr_ref, xs_ref):
    @pl.when(pl.program_id(0) == 0)
    def _init():
        for n in range(N_INPUT):
            wscr_ref[pl.ds(n * D, D), :] = wn_ref[n].astype(jnp.bfloat16)

    x_blk = x_ref[...]
    w_blk = w_ref[...]
    pidx_t = pidx_ref[...]
    p = p_ref[...]

    for n in range(N_INPUT):
        xs_ref[:, pl.ds(n * D, D)] = (x_blk * w_blk[:, n:n + 1]).astype(jnp.bfloat16)

    h0 = jnp.dot(xs_ref[...], wscr_ref[...], preferred_element_type=jnp.float32)
    out_ref[...] = h0

    vnorm = jnp.sum(p * p, axis=1, keepdims=True) + 1e-8
    vn = p * lax.rsqrt(vnorm)
    gn = lax.dot_general(vn, vn, (((1,), (1,)), ((), ())),
                         preferred_element_type=jnp.float32)

    d_t = lax.dot_general(vn, h0, (((1,), (1,)), ((), ())),
                          preferred_element_type=jnp.float32)
    coeff_t = jnp.zeros_like(d_t)
    ids = lax.broadcasted_iota(jnp.int32, (N_PROCESS, 1), 0)
    for k in range(K):
        onehot_t = (pidx_t[k:k + 1, :] == ids).astype(jnp.float32)
        c2 = 2.0 * jnp.sum(onehot_t * d_t, axis=0, keepdims=True)
        g_t = jnp.dot(gn, onehot_t, preferred_element_type=jnp.float32)
        d_t = d_t - c2 * g_t
        coeff_t = coeff_t + c2 * onehot_t

    corr = lax.dot_general(coeff_t, vn, (((0,), (0,)), ((), ())),
                           preferred_element_type=jnp.float32)
    out_ref[...] = out_ref[...] - corr


@jax.jit
def kernel(x, input_idx, input_weights, process_indices, input_neurons, process_neurons):
    del input_idx
    T = B * S
    xf = x.reshape(T, D)
    wf = input_weights.reshape(T, N_INPUT)
    pidx_t = process_indices.reshape(T, K).astype(jnp.int32).T

    grid = (T // T_BLK,)
    out = pl.pallas_call(
        _fused_kernel,
        grid=grid,
        in_specs=[
            pl.BlockSpec((T_BLK, D), lambda i: (i, 0)),
            pl.BlockSpec((T_BLK, N_INPUT), lambda i: (i, 0)),
            pl.BlockSpec((K, T_BLK), lambda i: (0, i)),
            pl.BlockSpec((N_INPUT, D, R), lambda i: (0, 0, 0)),
            pl.BlockSpec((N_PROCESS, R), lambda i: (0, 0)),
        ],
        out_specs=pl.BlockSpec((T_BLK, R), lambda i: (i, 0)),
        out_shape=jax.ShapeDtypeStruct((T, R), jnp.float32),
        scratch_shapes=[
            pltpu.VMEM((N_INPUT * D, R), jnp.bfloat16),
            pltpu.VMEM((T_BLK, N_INPUT * D), jnp.bfloat16),
        ],
    )(xf, wf, pidx_t, input_neurons, process_neurons)
    return out.reshape(B, S, R)


# speedup vs baseline: 1.0613x; 1.0613x over previous
import jax
import jax.numpy as jnp
from jax import lax
from jax.experimental import pallas as pl
from jax.experimental.pallas import tpu as pltpu

B, S, D, R, N_INPUT, N_PROCESS, K = 4, 2048, 2048, 256, 8, 32, 8
T_BLK = 1024
N_BLKS = (B * S) // T_BLK


def _fused_kernel(x_ref, w_ref, pidx_ref, wn_ref, p_ref, out_ref, wscr_ref, h0_ref):
    i = pl.program_id(0)

    @pl.when(i == 0)
    def _init():
        wscr_ref[...] = wn_ref[...].astype(jnp.bfloat16)

    @pl.when(i < N_BLKS)
    def _dense():
        x_blk = x_ref[...].astype(jnp.bfloat16)
        w_blk = w_ref[...]
        h0 = jnp.zeros((T_BLK, R), dtype=jnp.float32)
        for n in range(N_INPUT):
            proj = jnp.dot(x_blk, wscr_ref[n], preferred_element_type=jnp.float32)
            h0 = h0 + proj * w_blk[:, n:n + 1]
        h0_ref[i % 2] = h0

    @pl.when(i > 0)
    def _householder():
        h0 = h0_ref[(i + 1) % 2]
        pidx_t = pidx_ref[...]
        p = p_ref[...]
        vnorm = jnp.sum(p * p, axis=1, keepdims=True) + 1e-8
        vn = p * lax.rsqrt(vnorm)
        gn = lax.dot_general(vn, vn, (((1,), (1,)), ((), ())),
                             preferred_element_type=jnp.float32)

        d_t = lax.dot_general(vn, h0, (((1,), (1,)), ((), ())),
                              preferred_element_type=jnp.float32)
        coeff_t = jnp.zeros_like(d_t)
        ids = lax.broadcasted_iota(jnp.int32, (N_PROCESS, 1), 0)
        for k in range(K):
            onehot_t = (pidx_t[k:k + 1, :] == ids).astype(jnp.float32)
            c2 = 2.0 * jnp.sum(onehot_t * d_t, axis=0, keepdims=True)
            g_t = jnp.dot(gn, onehot_t, preferred_element_type=jnp.float32)
            d_t = d_t - c2 * g_t
            coeff_t = coeff_t + c2 * onehot_t

        corr = lax.dot_general(coeff_t, vn, (((0,), (0,)), ((), ())),
                               preferred_element_type=jnp.float32)
        out_ref[...] = h0 - corr


@jax.jit
def kernel(x, input_idx, input_weights, process_indices, input_neurons, process_neurons):
    del input_idx
    T = B * S
    xf = x.reshape(T, D)
    wf = input_weights.reshape(T, N_INPUT)
    pidx_t = process_indices.reshape(T, K).astype(jnp.int32).T

    last = N_BLKS - 1
    grid = (N_BLKS + 1,)
    out = pl.pallas_call(
        _fused_kernel,
        grid=grid,
        in_specs=[
            pl.BlockSpec((T_BLK, D), lambda i: (jnp.minimum(i, last), 0)),
            pl.BlockSpec((T_BLK, N_INPUT), lambda i: (jnp.minimum(i, last), 0)),
            pl.BlockSpec((K, T_BLK), lambda i: (0, jnp.maximum(i - 1, 0))),
            pl.BlockSpec((N_INPUT, D, R), lambda i: (0, 0, 0)),
            pl.BlockSpec((N_PROCESS, R), lambda i: (0, 0)),
        ],
        out_specs=pl.BlockSpec((T_BLK, R), lambda i: (jnp.maximum(i - 1, 0), 0)),
        out_shape=jax.ShapeDtypeStruct((T, R), jnp.float32),
        scratch_shapes=[
            pltpu.VMEM((N_INPUT, D, R), jnp.bfloat16),
            pltpu.VMEM((2, T_BLK, R), jnp.float32),
        ],
    )(xf, wf, pidx_t, input_neurons, process_neurons)
    return out.reshape(B, S, R)
